# Initial kernel scaffold; baseline (speedup 1.0000x reference)
import jax
import jax.numpy as jnp
from jax.experimental import pallas as pl


def kernel(x, edge_index, edge_type, W1, root1, b1, W2, root2, b2):
    raise NotImplementedError("write your pallas kernel here")


# speedup vs baseline: 16.1092x; 16.1092x over previous
import functools

import jax
import jax.numpy as jnp
from jax import lax
from jax.experimental import pallas as pl
from jax.experimental.pallas import tpu as pltpu
from jax.experimental.pallas import tpu_sc as plsc

N = 10000
E = 320000
D = 128
R = 8
RN = R * N

NC = 2
NS = 16
NW = NC * NS
EPW = E // NW
K = 80
CPT = RN // NS
N_PAD = 10240
ROWS_PT = N_PAD // NS
ZROWS = 40


def _sc_counts(widx):
    mesh = plsc.VectorSubcoreMesh(core_axis_name="c", subcore_axis_name="s")

    @functools.partial(
        pl.kernel,
        mesh=mesh,
        out_type=jax.ShapeDtypeStruct((NC * RN,), jnp.float32),
        scratch_types=[
            pltpu.VMEM_SHARED((RN,), jnp.float32),
            pltpu.VMEM((K,), jnp.int32),
            pltpu.VMEM((K,), jnp.float32),
            pltpu.VMEM((1000,), jnp.float32),
        ],
    )
    def body(widx_hbm, out_hbm, acc, idx_v, ones_v, zbuf_v):
        c = lax.axis_index("c")
        s = lax.axis_index("s")
        wid = c * NS + s
        one16 = jnp.full((16,), 1.0, jnp.float32)
        zero16 = jnp.zeros((16,), jnp.float32)
        for j in range(K // 16):
            ones_v[pl.ds(j * 16, 16)] = one16
        for j in range(1000 // 16):
            zbuf_v[pl.ds(j * 16, 16)] = zero16
        def zero_step(i, carry):
            pltpu.sync_copy(zbuf_v, acc.at[pl.ds(s * CPT + i * 1000, 1000)])
            return carry
        lax.fori_loop(0, CPT // 1000, zero_step, 0)
        plsc.subcore_barrier()
        base = wid * EPW
        def step(i, carry):
            pltpu.sync_copy(widx_hbm.at[pl.ds(base + i * K, K)], idx_v)
            pltpu.sync_copy(ones_v, acc.at[idx_v], add=True)
            return carry
        lax.fori_loop(0, EPW // K, step, 0)
        plsc.subcore_barrier()
        def wb_step(i, carry):
            off = s * CPT + i * 1000
            pltpu.sync_copy(acc.at[pl.ds(off, 1000)], zbuf_v)
            pltpu.sync_copy(zbuf_v, out_hbm.at[pl.ds(c * RN + off, 1000)])
            return carry
        lax.fori_loop(0, CPT // 1000, wb_step, 0)

    return body(widx)


def _sc_edge_pass(H, gidx, dst, widx, inv):
    mesh = plsc.VectorSubcoreMesh(core_axis_name="c", subcore_axis_name="s")

    @functools.partial(
        pl.kernel,
        mesh=mesh,
        out_type=jax.ShapeDtypeStruct((NC * N_PAD, D), jnp.float32),
        scratch_types=[
            pltpu.VMEM_SHARED((N_PAD, D), jnp.float32),
            pltpu.VMEM_SHARED((RN,), jnp.float32),
            pltpu.VMEM((1000,), jnp.float32),
            pltpu.VMEM((K,), jnp.int32),
            pltpu.VMEM((K,), jnp.int32),
            pltpu.VMEM((K,), jnp.int32),
            pltpu.VMEM((K, D), jnp.float32),
            pltpu.VMEM((K,), jnp.float32),
            pltpu.VMEM((ZROWS, D), jnp.float32),
            pltpu.SemaphoreType.DMA,
            pltpu.SemaphoreType.DMA,
        ],
    )
    def body(h_hbm, gidx_hbm, dst_hbm, widx_hbm, inv_hbm, out_hbm,
             acc, inv_sp, ibuf_v, gidx_v, dst_v, widx_v, rows_v, w_v,
             zbuf_v, sem, sem2):
        c = lax.axis_index("c")
        s = lax.axis_index("s")
        wid = c * NS + s
        zero16 = jnp.zeros((16,), jnp.float32)
        for i in range(ZROWS):
            for j in range(D // 16):
                zbuf_v[i, pl.ds(j * 16, 16)] = zero16
        def inv_step(i, carry):
            off = s * CPT + i * 1000
            pltpu.sync_copy(inv_hbm.at[pl.ds(off, 1000)], ibuf_v)
            pltpu.sync_copy(ibuf_v, inv_sp.at[pl.ds(off, 1000)])
            return carry
        lax.fori_loop(0, CPT // 1000, inv_step, 0)
        def zero_step(i, carry):
            pltpu.sync_copy(zbuf_v, acc.at[pl.ds(s * ROWS_PT + i * ZROWS, ZROWS)])
            return carry
        lax.fori_loop(0, ROWS_PT // ZROWS, zero_step, 0)
        plsc.subcore_barrier()

        base = wid * EPW
        def step(i, carry):
            b = base + i * K
            pltpu.sync_copy(gidx_hbm.at[pl.ds(b, K)], gidx_v)
            pltpu.sync_copy(dst_hbm.at[pl.ds(b, K)], dst_v)
            pltpu.sync_copy(widx_hbm.at[pl.ds(b, K)], widx_v)
            cp1 = pltpu.async_copy(h_hbm.at[gidx_v], rows_v, sem)
            cp2 = pltpu.async_copy(inv_sp.at[widx_v], w_v, sem2)
            cp1.wait()
            cp2.wait()
            def scale(g, carry2):
                w16 = w_v[pl.ds(g * 16, 16)]
                for kk in range(16):
                    w = zero16 + w16[kk]
                    k = g * 16 + kk
                    for j in range(D // 16):
                        rows_v[k, pl.ds(j * 16, 16)] = (
                            rows_v[k, pl.ds(j * 16, 16)] * w)
                return carry2
            lax.fori_loop(0, K // 16, scale, 0)
            pltpu.sync_copy(rows_v, acc.at[dst_v], add=True)
            return carry
        lax.fori_loop(0, EPW // K, step, 0)
        plsc.subcore_barrier()
        def wb_step(i, carry):
            r0 = s * ROWS_PT + i * ZROWS
            pltpu.sync_copy(acc.at[pl.ds(r0, ZROWS)], zbuf_v)
            pltpu.sync_copy(zbuf_v, out_hbm.at[pl.ds(c * N_PAD + r0, ZROWS)])
            return carry
        lax.fori_loop(0, ROWS_PT // ZROWS, wb_step, 0)

    return body(H, gidx, dst, widx, inv)


BN = 2000


def _tc_dense(x, wcat, brow):
    def body(x_ref, w_ref, b_ref, o_ref):
        r = pl.program_id(0)
        acc = jnp.dot(x_ref[...], w_ref[0],
                      preferred_element_type=jnp.float32)
        o_ref[0] = jnp.where(r == R, acc + b_ref[...], acc)

    return pl.pallas_call(
        body,
        grid=(R + 1, N // BN),
        in_specs=[
            pl.BlockSpec((BN, D), lambda r, i: (i, 0)),
            pl.BlockSpec((1, D, D), lambda r, i: (r, 0, 0)),
            pl.BlockSpec((1, D), lambda r, i: (0, 0)),
        ],
        out_specs=pl.BlockSpec((1, BN, D), lambda r, i: (r, i, 0)),
        out_shape=jax.ShapeDtypeStruct((R + 1, N, D), jnp.float32),
    )(x, wcat, brow)


def _tc_combine(base, p0, p1, do_relu):
    def body(b_ref, p0_ref, p1_ref, o_ref):
        v = b_ref[...] + p0_ref[...] + p1_ref[...]
        o_ref[...] = jnp.maximum(v, 0.0) if do_relu else v

    return pl.pallas_call(
        body,
        grid=(N // BN,),
        in_specs=[pl.BlockSpec((BN, D), lambda i: (i, 0))] * 3,
        out_specs=pl.BlockSpec((BN, D), lambda i: (i, 0)),
        out_shape=jax.ShapeDtypeStruct((N, D), jnp.float32),
    )(base, p0, p1)


def _layer(x, wcat, brow, gidx, dst, widx, inv, do_relu):
    hb = _tc_dense(x, wcat, brow)
    h_tab = hb[:R].reshape(RN, D)
    parts = _sc_edge_pass(h_tab, gidx, dst, widx, inv)
    return _tc_combine(hb[R], parts[:N], parts[N_PAD:N_PAD + N], do_relu)


def kernel(x, edge_index, edge_type, W1, root1, b1, W2, root2, b2):
    src = edge_index[0]
    dst = edge_index[1]
    gidx = edge_type * N + src
    widx = edge_type * N + dst
    counts2 = _sc_counts(widx)
    counts = counts2[:RN] + counts2[RN:]
    inv = 1.0 / jnp.maximum(counts, 1.0)
    wcat1 = jnp.concatenate([W1, root1[None]], axis=0)
    wcat2 = jnp.concatenate([W2, root2[None]], axis=0)
    h = _layer(x, wcat1, b1.reshape(1, D), gidx, dst, widx, inv, True)
    out = _layer(h, wcat2, b2.reshape(1, D), gidx, dst, widx, inv, False)
    return out
